# baseline (speedup 1.0000x reference)
import dataclasses

import jax
import jax.numpy as jnp
from jax import lax
from jax.experimental import pallas as pl
from jax.experimental.pallas import tpu as pltpu
from jax.experimental.pallas import tpu_sc as plsc

N_NODES = 10000
N_EDGES = 160000
D = 256
DH = 128
NC = 2
NS = 16
L = 16
K = 128
E_PER_SUB = 10240
NCHUNK = E_PER_SUB // K
PAD_E = E_PER_SUB - N_EDGES // NS
NP = 10240
R_PER_SUB = NP // NS
ZR = 128
PAD_DST = 10200
R_TC = 1024

_SC_COMPILER_PARAMS = pltpu.CompilerParams()
if "needs_layout_passes" in pltpu.CompilerParams.__dataclass_fields__:
    _SC_COMPILER_PARAMS = dataclasses.replace(
        _SC_COMPILER_PARAMS, needs_layout_passes=False)


def _make_sc_aggregate(need_w):
    mesh = plsc.VectorSubcoreMesh(core_axis_name="c", subcore_axis_name="s")

    def body(tab, ed3, *refs):
        if need_w:
            (out, out_w, acc, idx0, idx1, dstc0, dstc1, rows0, rows1,
             w_part, kbuf, cbuf,
             isem0, isem1, gsem0, gsem1, ssem0, ssem1) = refs
        else:
            (out, acc, idx0, idx1, dstc0, dstc1, rows0, rows1,
             isem0, isem1, gsem0, gsem1, ssem0, ssem1) = refs
        idx = (idx0, idx1)
        dstc = (dstc0, dstc1)
        rows = (rows0, rows1)
        isem = (isem0, isem1)
        gsem = (gsem0, gsem1)
        ssem = (ssem0, ssem1)
        c = lax.axis_index("c")
        s = lax.axis_index("s")
        iota = lax.iota(jnp.int32, L)
        off = jnp.full((L,), c * N_NODES, jnp.int32)

        @pl.loop(0, ZR)
        def _(r):
            for j in range(DH // L):
                rows0[r, pl.ds(j * L, L)] = jnp.zeros((L,), jnp.float32)

        for i in range(R_PER_SUB // ZR):
            pltpu.sync_copy(rows0, acc.at[pl.ds(s * R_PER_SUB + i * ZR, ZR)])

        if need_w:
            @pl.when(c == 0)
            def _():
                kbuf[pl.ds(L, L)] = jnp.full((L,), -1, jnp.int32)

                @pl.loop(0, NP // L)
                def _(r):
                    w_part[pl.ds(r * L, L)] = jnp.zeros((L,), jnp.float32)

        plsc.subcore_barrier()

        def stage_idx(k, b):
            pltpu.async_copy(ed3.at[s, k], idx[b], isem[b])

        def wait_idx(b):
            pltpu.make_async_copy(ed3.at[s, 0], idx[b], isem[b]).wait()

        def fire_gather(b):
            for g in range(K // L):
                sl = pl.ds(g * L, L)
                idx[b][0, sl] = idx[b][0, sl] + off
            pltpu.async_copy(tab.at[idx[b].at[0]], rows[b], gsem[b])

        def wait_gather(b):
            pltpu.make_async_copy(tab.at[idx[b].at[0]], rows[b],
                                  gsem[b]).wait()

        def consume(b):
            for g in range(K // L):
                sl = pl.ds(g * L, L)
                dstc[b][0, sl] = idx[b][1, sl]

            @plsc.parallel_loop(0, K, step=4, unroll=2)
            def _(e):
                for u in range(4):
                    ce = plsc.bitcast(
                        plsc.load_gather(
                            idx[b], [jnp.full((L,), 2, jnp.int32),
                                     jnp.full((L,), e + u, jnp.int32)]),
                        jnp.float32)
                    for j in range(DH // L):
                        sl = pl.ds(j * L, L)
                        rows[b][e + u, sl] = rows[b][e + u, sl] * ce

            if need_w:
                @pl.when(c == 0)
                def _():
                    for g in range(K // L):
                        sl = pl.ds(g * L, L)
                        d = idx[b][1, sl]
                        v = plsc.bitcast(idx[b][2, sl], jnp.float32)
                        ds_, vs_ = plsc.sort_key_val(d, v)
                        kbuf[pl.ds(0, L)] = ds_
                        knext = plsc.load_gather(kbuf, [iota + 1])
                        is_last = ds_ != knext
                        cum = plsc.cumsum(vs_)
                        cbuf[pl.ds(0, L)] = cum
                        kprev = plsc.load_gather(
                            kbuf, [jnp.maximum(iota - 1, 0)])
                        is_first = (ds_ != kprev) | (iota == 0)
                        start = plsc.cummax(jnp.where(is_first, iota, 0))
                        pc = plsc.load_gather(
                            cbuf, [jnp.maximum(start - 1, 0)])
                        prev = jnp.where(start == 0, 0.0, pc)
                        plsc.addupdate_scatter(
                            w_part, [ds_], cum - prev, mask=is_last)

        def fire_scatter(b):
            pltpu.async_copy(rows[b], acc.at[dstc[b].at[0]], ssem[b],
                             add=True)

        def wait_scatter(b):
            pltpu.make_async_copy(rows[b], acc.at[dstc[b].at[0]],
                                  ssem[b]).wait()

        stage_idx(0, 0)
        stage_idx(1, 1)
        wait_idx(0)
        fire_gather(0)

        @pl.loop(0, NCHUNK)
        def _(k):
            b = lax.rem(k, 2)

            @pl.when(k < NCHUNK - 1)
            def _():
                @pl.when(b == 0)
                def _():
                    wait_idx(1)

                @pl.when(b == 1)
                def _():
                    wait_idx(0)

            @pl.when(k >= 1)
            def _():
                @pl.when(b == 0)
                def _():
                    wait_scatter(1)

                @pl.when(b == 1)
                def _():
                    wait_scatter(0)

            @pl.when(k < NCHUNK - 1)
            def _():
                @pl.when(b == 0)
                def _():
                    fire_gather(1)

                @pl.when(b == 1)
                def _():
                    fire_gather(0)

            @pl.when(b == 0)
            def _():
                wait_gather(0)
                consume(0)
                fire_scatter(0)

                @pl.when(k < NCHUNK - 2)
                def _():
                    stage_idx(k + 2, 0)

            @pl.when(b == 1)
            def _():
                wait_gather(1)
                consume(1)
                fire_scatter(1)

                @pl.when(k < NCHUNK - 2)
                def _():
                    stage_idx(k + 2, 1)

        wait_scatter((NCHUNK - 1) % 2)
        plsc.subcore_barrier()
        for i in range(R_PER_SUB // ZR):
            r0 = s * R_PER_SUB + i * ZR
            pltpu.sync_copy(acc.at[pl.ds(r0, ZR)], rows0)
            pltpu.sync_copy(rows0, out.at[c, pl.ds(r0, ZR)])
        if need_w:
            @pl.when(c == 0)
            def _():
                pltpu.sync_copy(w_part, out_w.at[s])

    out_type = [jax.ShapeDtypeStruct((NC, NP, DH), jnp.float32)]
    if need_w:
        out_type.append(jax.ShapeDtypeStruct((NS, NP), jnp.float32))
    scratch = [
        pltpu.VMEM_SHARED((NP, DH), jnp.float32),
        pltpu.VMEM((3, K), jnp.int32),
        pltpu.VMEM((3, K), jnp.int32),
        pltpu.VMEM((1, K), jnp.int32),
        pltpu.VMEM((1, K), jnp.int32),
        pltpu.VMEM((K, DH), jnp.float32),
        pltpu.VMEM((K, DH), jnp.float32),
    ]
    if need_w:
        scratch += [
            pltpu.VMEM((NP,), jnp.float32),
            pltpu.VMEM((2 * L,), jnp.int32),
            pltpu.VMEM((L,), jnp.float32),
        ]
    scratch += [pltpu.SemaphoreType.DMA] * 6

    return pl.kernel(
        body,
        out_type=out_type,
        mesh=mesh,
        scratch_types=scratch,
        compiler_params=_SC_COMPILER_PARAMS,
    )


def _tc_pre(h, wb, b):
    def body(h_ref, wb_ref, b_ref, o_ref):
        o_ref[...] = jnp.dot(h_ref[...], wb_ref[...],
                             preferred_element_type=jnp.float32) + b_ref[0]

    f = pl.pallas_call(
        body,
        grid=(NP // R_TC,),
        in_specs=[
            pl.BlockSpec((R_TC, D), lambda i: (i, 0)),
            pl.BlockSpec((D, D), lambda i: (0, 0)),
            pl.BlockSpec((1, D), lambda i: (0, 0)),
        ],
        out_specs=pl.BlockSpec((R_TC, D), lambda i: (i, 0)),
        out_shape=jax.ShapeDtypeStruct((N_NODES, D), jnp.float32),
    )
    return f(h, wb, b)


def _tc_combine(g, w, pre, wt):
    def body(g0_ref, g1_ref, w_ref, p_ref, wt_ref, o_ref):
        wsum = jnp.sum(w_ref[...], axis=0)
        inv = 1.0 / jnp.maximum(wsum, 1.0)
        n0 = g0_ref[0] * inv[:, None]
        n1 = g1_ref[0] * inv[:, None]
        wtm = wt_ref[...]
        z = (jnp.dot(n0, wtm[:DH], preferred_element_type=jnp.float32)
             + jnp.dot(n1, wtm[DH:], preferred_element_type=jnp.float32)
             + p_ref[...])
        z = jnp.maximum(z, 0.0)
        nrm = jnp.sqrt(jnp.sum(z * z, axis=1, keepdims=True))
        nrm = jnp.where(nrm == 0.0, 1.0, nrm)
        o_ref[...] = z / nrm

    f = pl.pallas_call(
        body,
        grid=(NP // R_TC,),
        in_specs=[
            pl.BlockSpec((1, R_TC, DH), lambda i: (0, i, 0)),
            pl.BlockSpec((1, R_TC, DH), lambda i: (1, i, 0)),
            pl.BlockSpec((NS, R_TC), lambda i: (0, i)),
            pl.BlockSpec((R_TC, D), lambda i: (i, 0)),
            pl.BlockSpec((D, D), lambda i: (0, 0)),
        ],
        out_specs=pl.BlockSpec((R_TC, D), lambda i: (i, 0)),
        out_shape=jax.ShapeDtypeStruct((N_NODES, D), jnp.float32),
    )
    return f(g, g, w, pre, wt)


_sc_agg_w = _make_sc_aggregate(True)
_sc_agg = _make_sc_aggregate(False)


def _pack_edges(edge_index, edge_count):
    epr = N_EDGES // NS
    src = edge_index[0].astype(jnp.int32).reshape(NS, epr)
    dst = edge_index[1].astype(jnp.int32).reshape(NS, epr)
    cnt = edge_count.astype(jnp.float32).reshape(NS, epr)
    src = jnp.pad(src, ((0, 0), (0, PAD_E)))
    dst = jnp.pad(dst, ((0, 0), (0, PAD_E)), constant_values=PAD_DST)
    cnt = jnp.pad(cnt, ((0, 0), (0, PAD_E)))
    cnt_bits = lax.bitcast_convert_type(cnt, jnp.int32)
    ed3 = jnp.stack([src.reshape(NS, NCHUNK, K),
                     dst.reshape(NS, NCHUNK, K),
                     cnt_bits.reshape(NS, NCHUNK, K)], axis=2)
    return ed3


def kernel(x, edge_index, edge_count, W1, b1, W2, b2):
    ed3 = _pack_edges(edge_index, edge_count)

    tab1 = jnp.concatenate([x[:, :DH], x[:, DH:]], axis=0)
    agg1, w = _sc_agg_w(tab1, ed3)
    pre1 = _tc_pre(x, W1[D:], b1.reshape(1, D))
    h1 = _tc_combine(agg1, w, pre1, W1[:D])

    tab2 = jnp.concatenate([h1[:, :DH], h1[:, DH:]], axis=0)
    (agg2,) = _sc_agg(tab2, ed3)
    pre2 = _tc_pre(h1, W2[D:], b2.reshape(1, D))
    h2 = _tc_combine(agg2, w, pre2, W2[:D])
    return h2


# speedup vs baseline: 1.0147x; 1.0147x over previous
import dataclasses

import jax
import jax.numpy as jnp
from jax import lax
from jax.experimental import pallas as pl
from jax.experimental.pallas import tpu as pltpu
from jax.experimental.pallas import tpu_sc as plsc

N_NODES = 10000
N_EDGES = 160000
D = 256
DH = 128
NC = 2
NS = 16
L = 16
K = 128
E_PER_SUB = 10240
NCHUNK = E_PER_SUB // K
PAD_E = E_PER_SUB - N_EDGES // NS
NP = 10240
R_PER_SUB = NP // NS
ZR = 128
PAD_DST = 10200
R_TC = 1024

_SC_COMPILER_PARAMS = pltpu.CompilerParams()
if "needs_layout_passes" in pltpu.CompilerParams.__dataclass_fields__:
    _SC_COMPILER_PARAMS = dataclasses.replace(
        _SC_COMPILER_PARAMS, needs_layout_passes=False)


def _make_sc_aggregate(need_w):
    mesh = plsc.VectorSubcoreMesh(core_axis_name="c", subcore_axis_name="s")

    def body(tab, ed3, *refs):
        if need_w:
            (out, out_w, acc, idx0, idx1, dstc0, dstc1, rows0, rows1,
             w_part, kbuf, cbuf,
             isem0, isem1, gsem0, gsem1, ssem0, ssem1) = refs
        else:
            (out, acc, idx0, idx1, dstc0, dstc1, rows0, rows1,
             isem0, isem1, gsem0, gsem1, ssem0, ssem1) = refs
        idx = (idx0, idx1)
        dstc = (dstc0, dstc1)
        rows = (rows0, rows1)
        isem = (isem0, isem1)
        gsem = (gsem0, gsem1)
        ssem = (ssem0, ssem1)
        c = lax.axis_index("c")
        s = lax.axis_index("s")
        iota = lax.iota(jnp.int32, L)
        off = jnp.full((L,), c * N_NODES, jnp.int32)

        @pl.loop(0, ZR)
        def _(r):
            for j in range(DH // L):
                rows0[r, pl.ds(j * L, L)] = jnp.zeros((L,), jnp.float32)

        for i in range(R_PER_SUB // ZR):
            pltpu.sync_copy(rows0, acc.at[pl.ds(s * R_PER_SUB + i * ZR, ZR)])

        if need_w:
            @pl.when(c == 0)
            def _():
                kbuf[pl.ds(L, L)] = jnp.full((L,), -1, jnp.int32)

                @pl.loop(0, NP // L)
                def _(r):
                    w_part[pl.ds(r * L, L)] = jnp.zeros((L,), jnp.float32)

        plsc.subcore_barrier()

        def stage_idx(k, b):
            pltpu.async_copy(ed3.at[s, k], idx[b], isem[b])

        def wait_idx(b):
            pltpu.make_async_copy(ed3.at[s, 0], idx[b], isem[b]).wait()

        def fire_gather(b):
            for g in range(K // L):
                sl = pl.ds(g * L, L)
                idx[b][0, sl] = idx[b][0, sl] + off
            pltpu.async_copy(tab.at[idx[b].at[0]], rows[b], gsem[b])

        def wait_gather(b):
            pltpu.make_async_copy(tab.at[idx[b].at[0]], rows[b],
                                  gsem[b]).wait()

        def consume(b):
            for g in range(K // L):
                sl = pl.ds(g * L, L)
                dstc[b][0, sl] = idx[b][1, sl]

            @plsc.parallel_loop(0, K, step=4, unroll=2)
            def _(e):
                for u in range(4):
                    ce = plsc.bitcast(
                        plsc.load_gather(
                            idx[b], [jnp.full((L,), 2, jnp.int32),
                                     jnp.full((L,), e + u, jnp.int32)]),
                        jnp.float32)
                    for j in range(DH // L):
                        sl = pl.ds(j * L, L)
                        rows[b][e + u, sl] = rows[b][e + u, sl] * ce

            if need_w:
                @pl.when(c == 0)
                def _():
                    for g in range(K // L):
                        sl = pl.ds(g * L, L)
                        d = idx[b][1, sl]
                        v = plsc.bitcast(idx[b][2, sl], jnp.float32)
                        ds_, vs_ = plsc.sort_key_val(d, v)
                        kbuf[pl.ds(0, L)] = ds_
                        knext = plsc.load_gather(kbuf, [iota + 1])
                        is_last = ds_ != knext
                        cum = plsc.cumsum(vs_)
                        cbuf[pl.ds(0, L)] = cum
                        kprev = plsc.load_gather(
                            kbuf, [jnp.maximum(iota - 1, 0)])
                        is_first = (ds_ != kprev) | (iota == 0)
                        start = plsc.cummax(jnp.where(is_first, iota, 0))
                        pc = plsc.load_gather(
                            cbuf, [jnp.maximum(start - 1, 0)])
                        prev = jnp.where(start == 0, 0.0, pc)
                        plsc.addupdate_scatter(
                            w_part, [ds_], cum - prev, mask=is_last)

        def fire_scatter(b):
            pltpu.async_copy(rows[b], acc.at[dstc[b].at[0]], ssem[b],
                             add=True)

        def wait_scatter(b):
            pltpu.make_async_copy(rows[b], acc.at[dstc[b].at[0]],
                                  ssem[b]).wait()

        stage_idx(0, 0)
        stage_idx(1, 1)
        wait_idx(0)
        fire_gather(0)

        @pl.loop(0, NCHUNK)
        def _(k):
            b = lax.rem(k, 2)

            @pl.when(k < NCHUNK - 1)
            def _():
                @pl.when(b == 0)
                def _():
                    wait_idx(1)

                @pl.when(b == 1)
                def _():
                    wait_idx(0)

            @pl.when(k >= 1)
            def _():
                @pl.when(b == 0)
                def _():
                    wait_scatter(1)

                @pl.when(b == 1)
                def _():
                    wait_scatter(0)

            @pl.when(k < NCHUNK - 1)
            def _():
                @pl.when(b == 0)
                def _():
                    fire_gather(1)

                @pl.when(b == 1)
                def _():
                    fire_gather(0)

            @pl.when(b == 0)
            def _():
                wait_gather(0)
                consume(0)
                fire_scatter(0)

                @pl.when(k < NCHUNK - 2)
                def _():
                    stage_idx(k + 2, 0)

            @pl.when(b == 1)
            def _():
                wait_gather(1)
                consume(1)
                fire_scatter(1)

                @pl.when(k < NCHUNK - 2)
                def _():
                    stage_idx(k + 2, 1)

        wait_scatter((NCHUNK - 1) % 2)
        plsc.subcore_barrier()
        for i in range(R_PER_SUB // ZR):
            r0 = s * R_PER_SUB + i * ZR
            pltpu.sync_copy(acc.at[pl.ds(r0, ZR)], rows0)
            pltpu.sync_copy(rows0, out.at[c, pl.ds(r0, ZR)])
        if need_w:
            @pl.when(c == 0)
            def _():
                pltpu.sync_copy(w_part, out_w.at[s])

    out_type = [jax.ShapeDtypeStruct((NC, NP, DH), jnp.float32)]
    if need_w:
        out_type.append(jax.ShapeDtypeStruct((NS, NP), jnp.float32))
    scratch = [
        pltpu.VMEM_SHARED((NP, DH), jnp.float32),
        pltpu.VMEM((3, K), jnp.int32),
        pltpu.VMEM((3, K), jnp.int32),
        pltpu.VMEM((1, K), jnp.int32),
        pltpu.VMEM((1, K), jnp.int32),
        pltpu.VMEM((K, DH), jnp.float32),
        pltpu.VMEM((K, DH), jnp.float32),
    ]
    if need_w:
        scratch += [
            pltpu.VMEM((NP,), jnp.float32),
            pltpu.VMEM((2 * L,), jnp.int32),
            pltpu.VMEM((L,), jnp.float32),
        ]
    scratch += [pltpu.SemaphoreType.DMA] * 6

    return pl.kernel(
        body,
        out_type=out_type,
        mesh=mesh,
        scratch_types=scratch,
        compiler_params=_SC_COMPILER_PARAMS,
    )


def _dense(g, w, h, wm, b):
    def body(g0_ref, g1_ref, w_ref, h_ref, wm_ref, b_ref, o_ref):
        wsum = jnp.sum(w_ref[...], axis=0)
        inv = 1.0 / jnp.maximum(wsum, 1.0)
        n0 = g0_ref[0] * inv[:, None]
        n1 = g1_ref[0] * inv[:, None]
        wmat = wm_ref[...]
        z = (jnp.dot(n0, wmat[:DH], preferred_element_type=jnp.float32)
             + jnp.dot(n1, wmat[DH:2 * DH], preferred_element_type=jnp.float32)
             + jnp.dot(h_ref[...], wmat[2 * DH:],
                       preferred_element_type=jnp.float32))
        z = jnp.maximum(z + b_ref[0], 0.0)
        nrm = jnp.sqrt(jnp.sum(z * z, axis=1, keepdims=True))
        nrm = jnp.where(nrm == 0.0, 1.0, nrm)
        o_ref[...] = z / nrm

    nb = NP // R_TC
    f = pl.pallas_call(
        body,
        grid=(nb,),
        in_specs=[
            pl.BlockSpec((1, R_TC, DH), lambda i: (0, i, 0)),
            pl.BlockSpec((1, R_TC, DH), lambda i: (1, i, 0)),
            pl.BlockSpec((NS, R_TC), lambda i: (0, i)),
            pl.BlockSpec((R_TC, D), lambda i: (i, 0)),
            pl.BlockSpec((2 * D, D), lambda i: (0, 0)),
            pl.BlockSpec((1, D), lambda i: (0, 0)),
        ],
        out_specs=pl.BlockSpec((R_TC, D), lambda i: (i, 0)),
        out_shape=jax.ShapeDtypeStruct((N_NODES, D), jnp.float32),
    )
    return f(g, g, w, h, wm, b)


_sc_agg_w = _make_sc_aggregate(True)
_sc_agg = _make_sc_aggregate(False)


def _pack_edges(edge_index, edge_count):
    epr = N_EDGES // NS
    src = edge_index[0].astype(jnp.int32).reshape(NS, epr)
    dst = edge_index[1].astype(jnp.int32).reshape(NS, epr)
    cnt = edge_count.astype(jnp.float32).reshape(NS, epr)
    src = jnp.pad(src, ((0, 0), (0, PAD_E)))
    dst = jnp.pad(dst, ((0, 0), (0, PAD_E)), constant_values=PAD_DST)
    cnt = jnp.pad(cnt, ((0, 0), (0, PAD_E)))
    cnt_bits = lax.bitcast_convert_type(cnt, jnp.int32)
    ed3 = jnp.stack([src.reshape(NS, NCHUNK, K),
                     dst.reshape(NS, NCHUNK, K),
                     cnt_bits.reshape(NS, NCHUNK, K)], axis=2)
    return ed3


def kernel(x, edge_index, edge_count, W1, b1, W2, b2):
    ed3 = _pack_edges(edge_index, edge_count)

    tab1 = jnp.concatenate([x[:, :DH], x[:, DH:]], axis=0)
    agg1, w = _sc_agg_w(tab1, ed3)
    h1 = _dense(agg1, w, x, W1, b1.reshape(1, D))

    tab2 = jnp.concatenate([h1[:, :DH], h1[:, DH:]], axis=0)
    (agg2,) = _sc_agg(tab2, ed3)
    h2 = _dense(agg2, w, h1, W2, b2.reshape(1, D))
    return h2
